# Initial kernel scaffold; baseline (speedup 1.0000x reference)
import jax
import jax.numpy as jnp
from jax.experimental import pallas as pl


def kernel(x, Wq, Wk, Wv, Wf1, Wf2, Wbeta, Wog1, Wog2, norm_w, Wo):
    raise NotImplementedError("write your pallas kernel here")


# speedup vs baseline: 20.1072x; 20.1072x over previous
import functools

import jax
import jax.numpy as jnp
from jax.experimental import pallas as pl
from jax.experimental.pallas import tpu as pltpu

EPS = 1e-6
CT = 64
TB = 256
TBC = 256


def _silu(z):
    return z * jax.nn.sigmoid(z)


def _proj_kernel(H, HD, x_ref, wq_ref, wk_ref, wv_ref, wf1_ref, wf2_ref,
                 wb_ref, wog1_ref, wog2_ref,
                 q_ref, k_ref, kmb_ref, v_ref, lf_ref, g_ref):
    x = x_ref[0]
    f32 = jnp.float32
    q = _silu(jnp.dot(x, wq_ref[...], preferred_element_type=f32))
    kx = _silu(jnp.dot(x, wk_ref[...], preferred_element_type=f32))
    v = _silu(jnp.dot(x, wv_ref[...], preferred_element_type=f32))
    f = jnp.dot(jnp.dot(x, wf1_ref[...], preferred_element_type=f32),
                wf2_ref[...], preferred_element_type=f32)
    lf = jax.nn.log_sigmoid(f)
    beta = jax.nn.sigmoid(jnp.dot(x, wb_ref[...], preferred_element_type=f32)) * 2.0
    g = jax.nn.sigmoid(jnp.dot(jnp.dot(x, wog1_ref[...], preferred_element_type=f32),
                               wog2_ref[...], preferred_element_type=f32))
    for h in range(H):
        sl = slice(h * HD, (h + 1) * HD)
        kh = kx[:, sl]
        khn = kh / jnp.sqrt(jnp.sum(kh * kh, axis=-1, keepdims=True) + EPS)
        q_ref[h] = q[:, sl]
        k_ref[h] = khn
        kmb_ref[h] = -beta[:, h:h + 1] * khn
        v_ref[h] = v[:, sl]
        lf_ref[h] = lf[:, sl]
        g_ref[h] = g[:, sl]


def _scan_kernel(ct, hd, q_ref, k_ref, kmb_ref, v_ref, lf_ref, o_ref, s_ref):
    c = pl.program_id(1)

    @pl.when(c == 0)
    def _():
        s_ref[...] = jnp.zeros_like(s_ref)

    f32 = jnp.float32
    cc = 2 * ct
    lf = lf_ref[0]
    k = k_ref[0]
    kmb = kmb_ref[0]
    q = q_ref[0]
    v = v_ref[0]

    rt = jax.lax.broadcasted_iota(jnp.int32, (ct, ct), 0)
    ctco = jax.lax.broadcasted_iota(jnp.int32, (ct, ct), 1)
    tri = jnp.where(ctco <= rt, 1.0, 0.0).astype(f32)
    F = jnp.dot(tri, lf, preferred_element_type=f32)
    Fex = F - lf
    epF = jnp.exp(F)
    enF = jnp.exp(-F)
    epX = jnp.exp(Fex)
    enX = jnp.exp(-Fex)

    At = jnp.concatenate([kmb * enX, kmb * enF], axis=0)
    Bt = jnp.concatenate([k * epX, k * epF], axis=0)
    Kt = k * enF
    Qt = q * epF

    S = s_ref[...]

    def dot_tt(a, b):
        return jax.lax.dot_general(a, b, (((1,), (1,)), ((), ())),
                                   preferred_element_type=f32)

    def dot_ff(a, b):
        return jax.lax.dot_general(a, b, (((0,), (0,)), ((), ())),
                                   preferred_element_type=f32)

    r2 = jax.lax.broadcasted_iota(jnp.int32, (cc, cc), 0)
    c2 = jax.lax.broadcasted_iota(jnp.int32, (cc, cc), 1)
    tr = jnp.where(r2 < ct, 2 * r2, 2 * r2 - (cc - 1))
    tc = jnp.where(c2 < ct, 2 * c2, 2 * c2 - (cc - 1))

    LA = jnp.where(tc < tr, dot_tt(Bt, At), 0.0)
    rK = jax.lax.broadcasted_iota(jnp.int32, (cc, ct), 0)
    cK = jax.lax.broadcasted_iota(jnp.int32, (cc, ct), 1)
    trK = jnp.where(rK < ct, 2 * rK, 2 * rK - (cc - 1))
    LK = jnp.where(2 * cK + 1 < trK, dot_tt(Bt, Kt), 0.0)

    R = dot_tt(Bt, S) + jnp.dot(LK, v, preferred_element_type=f32)

    T = LA + jnp.where(r2 == c2, 1.0, 0.0)
    P = LA
    for _ in range(6):
        P = jnp.dot(P, P, preferred_element_type=f32)
        T = T + jnp.dot(T, P, preferred_element_type=f32)
    U = jnp.dot(T, R, preferred_element_type=f32)

    rO = jax.lax.broadcasted_iota(jnp.int32, (ct, cc), 0)
    cO = jax.lax.broadcasted_iota(jnp.int32, (ct, cc), 1)
    tcO = jnp.where(cO < ct, 2 * cO, 2 * cO - (cc - 1))
    QA = jnp.where(tcO <= 2 * rO + 1, dot_tt(Qt, At), 0.0)
    QK = jnp.where(ctco <= rt, dot_tt(Qt, Kt), 0.0)
    O = (dot_tt(Qt, S) + jnp.dot(QA, U, preferred_element_type=f32)
         + jnp.dot(QK, v, preferred_element_type=f32))
    o_ref[0] = O

    gam = epF[ct - 1:ct, :]
    s_ref[...] = (S + dot_ff(U, At) + dot_ff(v, Kt)) * gam


def _out_kernel(H, HD, o_ref, g_ref, nw_ref, wo_ref, y_ref):
    cols = []
    for h in range(H):
        yh = o_ref[h] * g_ref[h]
        yh = yh / jnp.sqrt(jnp.mean(yh * yh, axis=-1, keepdims=True) + EPS)
        cols.append(yh * nw_ref[h:h + 1, :])
    y = jnp.concatenate(cols, axis=-1)
    y_ref[0] = jnp.dot(y, wo_ref[...], preferred_element_type=jnp.float32)


def kernel(x, Wq, Wk, Wv, Wf1, Wf2, Wbeta, Wog1, Wog2, norm_w, Wo):
    B, N, D = x.shape
    H = Wbeta.shape[1]
    HD = D // H
    BH = B * H
    nt = N // TB
    f32 = jnp.float32
    wspec = pl.BlockSpec(memory_space=pltpu.VMEM)
    hspec = pl.BlockSpec((H, TB, HD), lambda b, t: (b, t, 0))
    sds = jax.ShapeDtypeStruct((BH, N, HD), f32)

    q, k, kmb, v, lf, g = pl.pallas_call(
        functools.partial(_proj_kernel, H, HD),
        grid=(B, nt),
        in_specs=[pl.BlockSpec((1, TB, D), lambda b, t: (b, t, 0))]
        + [wspec] * 8,
        out_specs=[hspec] * 6,
        out_shape=[sds] * 6,
        compiler_params=pltpu.CompilerParams(
            dimension_semantics=("parallel", "parallel"),
            vmem_limit_bytes=60 * 1024 * 1024,
        ),
        name="dense_rnn_proj",
    )(x, Wq, Wk, Wv, Wf1, Wf2, Wbeta, Wog1, Wog2)

    nc = N // CT
    cspec = pl.BlockSpec((1, CT, HD), lambda bh, c: (bh, c, 0))
    o = pl.pallas_call(
        functools.partial(_scan_kernel, CT, HD),
        grid=(BH, nc),
        in_specs=[cspec] * 5,
        out_specs=cspec,
        out_shape=sds,
        scratch_shapes=[pltpu.VMEM((HD, HD), f32)],
        compiler_params=pltpu.CompilerParams(
            dimension_semantics=("parallel", "arbitrary"),
            vmem_limit_bytes=60 * 1024 * 1024,
        ),
        name="dense_rnn_scan",
    )(q, k, kmb, v, lf)

    ntc = N // TBC
    y = pl.pallas_call(
        functools.partial(_out_kernel, H, HD),
        grid=(B, ntc),
        in_specs=[pl.BlockSpec((H, TBC, HD), lambda b, t: (b, t, 0))] * 2
        + [wspec, wspec],
        out_specs=pl.BlockSpec((1, TBC, D), lambda b, t: (b, t, 0)),
        out_shape=jax.ShapeDtypeStruct((B, N, D), f32),
        compiler_params=pltpu.CompilerParams(
            dimension_semantics=("parallel", "parallel"),
            vmem_limit_bytes=60 * 1024 * 1024,
        ),
        name="dense_rnn_out",
    )(o, g, norm_w.reshape(H, HD), Wo)
    return y


# speedup vs baseline: 21.3730x; 1.0630x over previous
import functools

import jax
import jax.numpy as jnp
from jax.experimental import pallas as pl
from jax.experimental.pallas import tpu as pltpu

EPS = 1e-6
CT = 64
G = 4
TB = 256
TBC = 256


def _silu(z):
    return z * jax.nn.sigmoid(z)


def _proj_kernel(H, HD, x_ref, wq_ref, wk_ref, wv_ref, wf1_ref, wf2_ref,
                 wb_ref, wog1_ref, wog2_ref,
                 q_ref, k_ref, kmb_ref, v_ref, lf_ref, g_ref):
    x = x_ref[0]
    f32 = jnp.float32
    q = _silu(jnp.dot(x, wq_ref[...], preferred_element_type=f32))
    kx = _silu(jnp.dot(x, wk_ref[...], preferred_element_type=f32))
    v = _silu(jnp.dot(x, wv_ref[...], preferred_element_type=f32))
    f = jnp.dot(jnp.dot(x, wf1_ref[...], preferred_element_type=f32),
                wf2_ref[...], preferred_element_type=f32)
    lf = jax.nn.log_sigmoid(f)
    beta = jax.nn.sigmoid(jnp.dot(x, wb_ref[...], preferred_element_type=f32)) * 2.0
    g = jax.nn.sigmoid(jnp.dot(jnp.dot(x, wog1_ref[...], preferred_element_type=f32),
                               wog2_ref[...], preferred_element_type=f32))
    for h in range(H):
        sl = slice(h * HD, (h + 1) * HD)
        kh = kx[:, sl]
        khn = kh / jnp.sqrt(jnp.sum(kh * kh, axis=-1, keepdims=True) + EPS)
        q_ref[h] = q[:, sl]
        k_ref[h] = khn
        kmb_ref[h] = -beta[:, h:h + 1] * khn
        v_ref[h] = v[:, sl]
        lf_ref[h] = lf[:, sl]
        g_ref[h] = g[:, sl]


def _scan_kernel(ct, hd, g, q_ref, k_ref, kmb_ref, v_ref, lf_ref, o_ref, s_ref):
    c = pl.program_id(1)

    @pl.when(c == 0)
    def _():
        s_ref[...] = jnp.zeros_like(s_ref)

    f32 = jnp.float32
    cc = 2 * ct

    rt = jax.lax.broadcasted_iota(jnp.int32, (ct, ct), 0)
    ctco = jax.lax.broadcasted_iota(jnp.int32, (ct, ct), 1)
    tri = jnp.where(ctco <= rt, 1.0, 0.0).astype(f32)
    r2 = jax.lax.broadcasted_iota(jnp.int32, (cc, cc), 0)
    c2 = jax.lax.broadcasted_iota(jnp.int32, (cc, cc), 1)
    tr = jnp.where(r2 < ct, 2 * r2, 2 * r2 - (cc - 1))
    tc = jnp.where(c2 < ct, 2 * c2, 2 * c2 - (cc - 1))
    mask_la = tc < tr
    eye = jnp.where(r2 == c2, 1.0, 0.0)
    rK = jax.lax.broadcasted_iota(jnp.int32, (cc, ct), 0)
    cK = jax.lax.broadcasted_iota(jnp.int32, (cc, ct), 1)
    trK = jnp.where(rK < ct, 2 * rK, 2 * rK - (cc - 1))
    mask_lk = 2 * cK + 1 < trK
    rO = jax.lax.broadcasted_iota(jnp.int32, (ct, cc), 0)
    cO = jax.lax.broadcasted_iota(jnp.int32, (ct, cc), 1)
    tcO = jnp.where(cO < ct, 2 * cO, 2 * cO - (cc - 1))
    mask_qa = tcO <= 2 * rO + 1
    mask_qk = ctco <= rt

    def dot_tt(a, b):
        return jax.lax.dot_general(a, b, (((1,), (1,)), ((), ())),
                                   preferred_element_type=f32)

    def dot_ff(a, b):
        return jax.lax.dot_general(a, b, (((0,), (0,)), ((), ())),
                                   preferred_element_type=f32)

    def one_head(gi):
        lf = lf_ref[gi]
        k = k_ref[gi]
        kmb = kmb_ref[gi]
        q = q_ref[gi]
        v = v_ref[gi]

        F = jnp.dot(tri, lf, preferred_element_type=f32)
        Fex = F - lf
        epF = jnp.exp(F)
        enF = jnp.exp(-F)
        epX = jnp.exp(Fex)
        enX = jnp.exp(-Fex)

        At = jnp.concatenate([kmb * enX, kmb * enF], axis=0)
        Bt = jnp.concatenate([k * epX, k * epF], axis=0)
        Kt = k * enF
        Qt = q * epF

        S = s_ref[gi]

        LA = jnp.where(mask_la, dot_tt(Bt, At), 0.0)
        LK = jnp.where(mask_lk, dot_tt(Bt, Kt), 0.0)
        R = dot_tt(Bt, S) + jnp.dot(LK, v, preferred_element_type=f32)

        T = LA + eye
        P = LA
        for _ in range(6):
            P = jnp.dot(P, P, preferred_element_type=f32)
            T = T + jnp.dot(T, P, preferred_element_type=f32)
        U = jnp.dot(T, R, preferred_element_type=f32)

        QA = jnp.where(mask_qa, dot_tt(Qt, At), 0.0)
        QK = jnp.where(mask_qk, dot_tt(Qt, Kt), 0.0)
        O = (dot_tt(Qt, S) + jnp.dot(QA, U, preferred_element_type=f32)
             + jnp.dot(QK, v, preferred_element_type=f32))
        o_ref[gi] = O

        gam = epF[ct - 1:ct, :]
        s_ref[gi] = (S + dot_ff(U, At) + dot_ff(v, Kt)) * gam

    for gi in range(g):
        one_head(gi)


def _out_kernel(H, HD, o_ref, g_ref, nw_ref, wo_ref, y_ref):
    cols = []
    for h in range(H):
        yh = o_ref[h] * g_ref[h]
        yh = yh / jnp.sqrt(jnp.mean(yh * yh, axis=-1, keepdims=True) + EPS)
        cols.append(yh * nw_ref[h:h + 1, :])
    y = jnp.concatenate(cols, axis=-1)
    y_ref[0] = jnp.dot(y, wo_ref[...], preferred_element_type=jnp.float32)


def kernel(x, Wq, Wk, Wv, Wf1, Wf2, Wbeta, Wog1, Wog2, norm_w, Wo):
    B, N, D = x.shape
    H = Wbeta.shape[1]
    HD = D // H
    BH = B * H
    nt = N // TB
    f32 = jnp.float32
    wspec = pl.BlockSpec(memory_space=pltpu.VMEM)
    hspec = pl.BlockSpec((H, TB, HD), lambda b, t: (b, t, 0))
    sds = jax.ShapeDtypeStruct((BH, N, HD), f32)

    q, k, kmb, v, lf, g = pl.pallas_call(
        functools.partial(_proj_kernel, H, HD),
        grid=(B, nt),
        in_specs=[pl.BlockSpec((1, TB, D), lambda b, t: (b, t, 0))]
        + [wspec] * 8,
        out_specs=[hspec] * 6,
        out_shape=[sds] * 6,
        compiler_params=pltpu.CompilerParams(
            dimension_semantics=("parallel", "parallel"),
            vmem_limit_bytes=60 * 1024 * 1024,
        ),
        name="dense_rnn_proj",
    )(x, Wq, Wk, Wv, Wf1, Wf2, Wbeta, Wog1, Wog2)

    nc = N // CT
    cspec = pl.BlockSpec((G, CT, HD), lambda bh, c: (bh, c, 0))
    o = pl.pallas_call(
        functools.partial(_scan_kernel, CT, HD, G),
        grid=(BH // G, nc),
        in_specs=[cspec] * 5,
        out_specs=cspec,
        out_shape=sds,
        scratch_shapes=[pltpu.VMEM((G, HD, HD), f32)],
        compiler_params=pltpu.CompilerParams(
            dimension_semantics=("parallel", "arbitrary"),
            vmem_limit_bytes=60 * 1024 * 1024,
        ),
        name="dense_rnn_scan",
    )(q, k, kmb, v, lf)

    ntc = N // TBC
    y = pl.pallas_call(
        functools.partial(_out_kernel, H, HD),
        grid=(B, ntc),
        in_specs=[pl.BlockSpec((H, TBC, HD), lambda b, t: (b, t, 0))] * 2
        + [wspec, wspec],
        out_specs=pl.BlockSpec((1, TBC, D), lambda b, t: (b, t, 0)),
        out_shape=jax.ShapeDtypeStruct((B, N, D), f32),
        compiler_params=pltpu.CompilerParams(
            dimension_semantics=("parallel", "parallel"),
            vmem_limit_bytes=60 * 1024 * 1024,
        ),
        name="dense_rnn_out",
    )(o, g, norm_w.reshape(H, HD), Wo)
    return y
